# baseline (speedup 1.0000x reference)
import jax
import jax.numpy as jnp
from jax import lax
from jax.experimental import pallas as pl
from jax.experimental.pallas import tpu as pltpu
from jax.experimental.pallas import tpu_sc as plsc

MAX_SIZE = 38
NODE_DIM = 9
BOND_DIM = 4
N_EDGES = 378
B = 16384
NODE_W = MAX_SIZE * NODE_DIM
EDGE_W = N_EDGES * BOND_DIM
NODE_WP = 352

NW = 32
ROWS_W = B // NW
GROUPS = ROWS_W // 16

_LN2 = 0.6931471805599453
_SQRTH = 0.7071067811865476
_C3 = 2.0 / 3.0
_C5 = 2.0 / 5.0
_C7 = 2.0 / 7.0


def _log(u):
    bits = lax.bitcast_convert_type(u, jnp.int32)
    e = (bits >> 23) - 126
    m = lax.bitcast_convert_type(
        (bits & 0x007FFFFF) | 0x3F000000, jnp.float32)
    cond = m < _SQRTH
    m = jnp.where(cond, m + m, m)
    ef = (e - cond.astype(jnp.int32)).astype(jnp.float32)
    r = (m - 1.0) / (m + 1.0)
    r2 = r * r
    w = ((_C7 * r2 + _C5) * r2 + _C3) * r2 + 2.0
    return ef * _LN2 + r * w


def _argmin_step(j, s, best, bj):
    lt = s < best
    return jnp.where(lt, s, best), jnp.where(lt, jnp.int32(j), bj)


def _body(un, ue, nnegc, nlp, enegc, elp,
          out_lp, out_noh, out_eoh,
          ubn0, ubn1, ube0, ube1, ohn0, ohn1, ohe0, ohe1,
          tnc, tnl, tec, tel, acc0, acc1, sems):
    wid = lax.axis_index("s") * 2 + lax.axis_index("c")

    pltpu.sync_copy(nnegc, tnc)
    pltpu.sync_copy(nlp, tnl)
    pltpu.sync_copy(enegc, tec)
    pltpu.sync_copy(elp, tel)

    lane = lax.iota(jnp.int32, 16)
    base_n = lane * NODE_W
    base_e = lane * EDGE_W

    slots = ((ubn0, ube0, ohn0, ohe0, acc0, 0),
             (ubn1, ube1, ohn1, ohe1, acc1, 1))

    def in_copies(g, slot):
        ubn, ube = slots[slot][0], slots[slot][1]
        r0 = wid * ROWS_W + g * 16
        return (pltpu.make_async_copy(un.at[pl.ds(r0 * NODE_W, 16 * NODE_W)],
                                      ubn, sems.at[slot, 0]),
                pltpu.make_async_copy(ue.at[pl.ds(r0 * EDGE_W, 16 * EDGE_W)],
                                      ube, sems.at[slot, 1]))

    def out_copies(g, slot):
        _, _, ohn, ohe, acc, _ = slots[slot]
        r0 = wid * ROWS_W + g * 16
        return (pltpu.make_async_copy(ohn, out_noh.at[pl.ds(r0 * NODE_W, 16 * NODE_W)],
                                      sems.at[slot, 2]),
                pltpu.make_async_copy(ohe, out_eoh.at[pl.ds(r0 * EDGE_W, 16 * EDGE_W)],
                                      sems.at[slot, 3]),
                pltpu.make_async_copy(acc, out_lp.at[pl.ds(r0, 16)],
                                      sems.at[slot, 4]))

    def start_in(g, slot):
        for c in in_copies(g, slot):
            c.start()

    def wait_in(g, slot):
        for c in in_copies(g, slot):
            c.wait()

    def start_out(g, slot):
        for c in out_copies(g, slot):
            c.start()

    def wait_out(g, slot):
        for c in out_copies(g, slot):
            c.wait()

    def compute(g, slot):
        ubn, ube, ohn, ohe, accb, _ = slots[slot]

        def node_pos(i, acc):
            off = i * NODE_DIM
            best = jnp.full((16,), jnp.float32(jnp.inf))
            bj = jnp.zeros((16,), jnp.int32)
            for j in range(NODE_DIM):
                u = plsc.load_gather(ubn, [base_n + (off + j)])
                u = jnp.maximum(u, 1e-10)
                negc = plsc.load_gather(tnc, [jnp.full((16,), off + j, jnp.int32)])
                s = _log(u) * negc
                best, bj = _argmin_step(j, s, best, bj)
            ll = plsc.load_gather(tnl, [off + bj])
            for j in range(NODE_DIM):
                plsc.store_scatter(ohn, [base_n + (off + j)],
                                   (bj == j).astype(jnp.float32))
            return acc + ll

        acc = plsc.parallel_loop(0, MAX_SIZE, unroll=2,
                                 carry=jnp.zeros((16,), jnp.float32))(node_pos)

        def edge_pos(e, acc):
            off = e * BOND_DIM
            best = jnp.full((16,), jnp.float32(jnp.inf))
            bj = jnp.zeros((16,), jnp.int32)
            for j in range(BOND_DIM):
                u = plsc.load_gather(ube, [base_e + (off + j)])
                u = jnp.maximum(u, 1e-10)
                negc = plsc.load_gather(tec, [jnp.full((16,), off + j, jnp.int32)])
                s = _log(u) * negc
                best, bj = _argmin_step(j, s, best, bj)
            ll = plsc.load_gather(tel, [off + bj])
            for j in range(BOND_DIM):
                plsc.store_scatter(ohe, [base_e + (off + j)],
                                   (bj == j).astype(jnp.float32))
            return acc + ll

        acc = plsc.parallel_loop(0, N_EDGES, unroll=6, carry=acc)(edge_pos)
        accb[...] = acc

    start_in(0, 0)

    def pair(p, _):
        for k in range(2):
            g = p * 2 + k

            @pl.when(g + 1 < GROUPS)
            def _():
                start_in(g + 1, 1 - k)

            wait_in(g, k)

            @pl.when(g >= 2)
            def _():
                wait_out(g - 2, k)

            compute(g, k)
            start_out(g, k)
        return 0

    lax.fori_loop(0, GROUPS // 2, pair, 0)
    wait_out(GROUPS - 2, 0)
    wait_out(GROUPS - 1, 1)


@jax.jit
def kernel(u_node, u_edge, node_base_log_probs, edge_base_log_probs):
    nl = node_base_log_probs * 0.3
    el = edge_base_log_probs / 0.3
    n_negc = jnp.pad(-jnp.exp(-nl).reshape(-1), (0, NODE_WP - NODE_W),
                     constant_values=-1.0)
    n_lp = jnp.pad(jax.nn.log_softmax(nl, axis=-1).reshape(-1),
                   (0, NODE_WP - NODE_W))
    e_negc = -jnp.exp(-el).reshape(-1)
    e_lp = jax.nn.log_softmax(el, axis=-1).reshape(-1)

    mesh = plsc.VectorSubcoreMesh(core_axis_name="c", subcore_axis_name="s")
    call = pl.kernel(
        _body,
        out_type=[
            jax.ShapeDtypeStruct((B,), jnp.float32),
            jax.ShapeDtypeStruct((B * NODE_W,), jnp.float32),
            jax.ShapeDtypeStruct((B * EDGE_W,), jnp.float32),
        ],
        mesh=mesh,
        compiler_params=pltpu.CompilerParams(needs_layout_passes=False),
        scratch_types=[
            pltpu.VMEM((16 * NODE_W,), jnp.float32),
            pltpu.VMEM((16 * NODE_W,), jnp.float32),
            pltpu.VMEM((16 * EDGE_W,), jnp.float32),
            pltpu.VMEM((16 * EDGE_W,), jnp.float32),
            pltpu.VMEM((16 * NODE_W,), jnp.float32),
            pltpu.VMEM((16 * NODE_W,), jnp.float32),
            pltpu.VMEM((16 * EDGE_W,), jnp.float32),
            pltpu.VMEM((16 * EDGE_W,), jnp.float32),
            pltpu.VMEM((NODE_WP,), jnp.float32),
            pltpu.VMEM((NODE_WP,), jnp.float32),
            pltpu.VMEM((EDGE_W,), jnp.float32),
            pltpu.VMEM((EDGE_W,), jnp.float32),
            pltpu.VMEM((16,), jnp.float32),
            pltpu.VMEM((16,), jnp.float32),
            pltpu.SemaphoreType.DMA((2, 5)),
        ],
    )
    tlp, noh, eoh = call(u_node.reshape(-1), u_edge.reshape(-1),
                         n_negc, n_lp, e_negc, e_lp)
    return (tlp,
            noh.reshape(B, MAX_SIZE, NODE_DIM),
            eoh.reshape(B, N_EDGES, BOND_DIM))


# speedup vs baseline: 2.5485x; 2.5485x over previous
import jax
import jax.numpy as jnp
from jax import lax
from jax.experimental import pallas as pl
from jax.experimental.pallas import tpu as pltpu
from jax.experimental.pallas import tpu_sc as plsc

MAX_SIZE = 38
NODE_DIM = 9
BOND_DIM = 4
N_EDGES = 378
B = 16384
NODE_W = MAX_SIZE * NODE_DIM
EDGE_W = N_EDGES * BOND_DIM
NODE_WP = 352

NW = 32
ROWS_W = B // NW
GROUPS = ROWS_W // 16


_LN2 = 0.6931471805599453
_SQRTH = 0.7071067811865476
_C3 = 2.0 / 3.0
_C5 = 2.0 / 5.0
_C7 = 2.0 / 7.0


def _log(u):
    bits = lax.bitcast_convert_type(u, jnp.int32)
    e = (bits >> 23) - 126
    m = lax.bitcast_convert_type(
        (bits & 0x007FFFFF) | 0x3F000000, jnp.float32)
    cond = m < _SQRTH
    m = jnp.where(cond, m + m, m)
    ef = (e - cond.astype(jnp.int32)).astype(jnp.float32)
    r = (m - 1.0) / (m + 1.0)
    r2 = r * r
    w = ((_C7 * r2 + _C5) * r2 + _C3) * r2 + 2.0
    return ef * _LN2 + r * w


def _argmin_step(j, s, best, bj):
    lt = s < best
    return jnp.where(lt, s, best), jnp.where(lt, jnp.int32(j), bj)


def _body(un, ue, nnegc, nlp, enegc, elp,
          out_lp, out_noh, out_eoh,
          ubn0, ubn1, ube0, ube1, ohn0, ohn1, ohe0, ohe1,
          tnc, tnl, tec, tel, acc0, acc1, sems):
    wid = lax.axis_index("s") * 2 + lax.axis_index("c")


    pltpu.sync_copy(nnegc, tnc)
    pltpu.sync_copy(nlp, tnl)
    pltpu.sync_copy(enegc, tec)
    pltpu.sync_copy(elp, tel)

    lane = lax.iota(jnp.int32, 16)
    base_n = lane * NODE_W
    base_e = lane * EDGE_W

    slots = ((ubn0, ube0, ohn0, ohe0, acc0, 0),
             (ubn1, ube1, ohn1, ohe1, acc1, 1))

    def in_copies(g, slot):
        ubn, ube = slots[slot][0], slots[slot][1]
        r0 = wid * ROWS_W + g * 16
        return (pltpu.make_async_copy(un.at[pl.ds(r0 * NODE_W, 16 * NODE_W)],
                                      ubn, sems.at[slot, 0]),
                pltpu.make_async_copy(ue.at[pl.ds(r0 * EDGE_W, 16 * EDGE_W)],
                                      ube, sems.at[slot, 1]))

    def out_copies(g, slot):
        _, _, ohn, ohe, acc, _ = slots[slot]
        r0 = wid * ROWS_W + g * 16
        return (pltpu.make_async_copy(ohn, out_noh.at[pl.ds(r0 * NODE_W, 16 * NODE_W)],
                                      sems.at[slot, 2]),
                pltpu.make_async_copy(ohe, out_eoh.at[pl.ds(r0 * EDGE_W, 16 * EDGE_W)],
                                      sems.at[slot, 3]),
                pltpu.make_async_copy(acc, out_lp.at[pl.ds(r0, 16)],
                                      sems.at[slot, 4]))

    def start_in(g, slot):
        for c in in_copies(g, slot):
            c.start()

    def wait_in(g, slot):
        for c in in_copies(g, slot):
            c.wait()

    def start_out(g, slot):
        for c in out_copies(g, slot):
            c.start()

    def wait_out(g, slot):
        for c in out_copies(g, slot):
            c.wait()

    def compute(g, slot):
        ubn, ube, ohn, ohe, accb, _ = slots[slot]

        def node_pos(i, acc):
            off = i * NODE_DIM
            best = jnp.full((16,), jnp.float32(jnp.inf))
            bj = jnp.zeros((16,), jnp.int32)
            for j in range(NODE_DIM):
                u = plsc.load_gather(ubn, [base_n + (j * MAX_SIZE + i)])
                u = jnp.maximum(u, 1e-10)
                negc = plsc.load_gather(tnc, [jnp.full((16,), off + j, jnp.int32)])
                s = _log(u) * negc
                best, bj = _argmin_step(j, s, best, bj)
            ll = plsc.load_gather(tnl, [off + bj])
            for j in range(NODE_DIM):
                plsc.store_scatter(ohn, [base_n + (off + j)],
                                   (bj == j).astype(jnp.float32))
            return acc + ll

        acc = plsc.parallel_loop(0, MAX_SIZE, unroll=2,
                                 carry=jnp.zeros((16,), jnp.float32))(node_pos)

        def edge_pos(e, acc):
            off = e * BOND_DIM
            best = jnp.full((16,), jnp.float32(jnp.inf))
            bj = jnp.zeros((16,), jnp.int32)
            for j in range(BOND_DIM):
                u = plsc.load_gather(ube, [base_e + (j * N_EDGES + e)])
                u = jnp.maximum(u, 1e-10)
                negc = plsc.load_gather(tec, [jnp.full((16,), off + j, jnp.int32)])
                s = _log(u) * negc
                best, bj = _argmin_step(j, s, best, bj)
            ll = plsc.load_gather(tel, [off + bj])
            for j in range(BOND_DIM):
                plsc.store_scatter(ohe, [base_e + (off + j)],
                                   (bj == j).astype(jnp.float32))
            return acc + ll

        acc = plsc.parallel_loop(0, N_EDGES, unroll=6, carry=acc)(edge_pos)
        accb[...] = acc

    start_in(0, 0)

    def pair(p, _):
        for k in range(2):
            g = p * 2 + k

            @pl.when(g + 1 < GROUPS)
            def _():
                start_in(g + 1, 1 - k)

            wait_in(g, k)

            @pl.when(g >= 2)
            def _():
                wait_out(g - 2, k)

            compute(g, k)
            start_out(g, k)
        return 0

    lax.fori_loop(0, GROUPS // 2, pair, 0)
    wait_out(GROUPS - 2, 0)
    wait_out(GROUPS - 1, 1)


@jax.jit
def kernel(u_node, u_edge, node_base_log_probs, edge_base_log_probs):
    nl = node_base_log_probs * 0.3
    el = edge_base_log_probs / 0.3
    n_negc = jnp.pad(-jnp.exp(-nl).reshape(-1), (0, NODE_WP - NODE_W),
                     constant_values=-1.0)
    n_lp = jnp.pad(jax.nn.log_softmax(nl, axis=-1).reshape(-1),
                   (0, NODE_WP - NODE_W))
    e_negc = -jnp.exp(-el).reshape(-1)
    e_lp = jax.nn.log_softmax(el, axis=-1).reshape(-1)

    mesh = plsc.VectorSubcoreMesh(core_axis_name="c", subcore_axis_name="s")
    call = pl.kernel(
        _body,
        out_type=[
            jax.ShapeDtypeStruct((B,), jnp.float32),
            jax.ShapeDtypeStruct((B * NODE_W,), jnp.float32),
            jax.ShapeDtypeStruct((B * EDGE_W,), jnp.float32),
        ],
        mesh=mesh,
        compiler_params=pltpu.CompilerParams(needs_layout_passes=False),
        scratch_types=[
            pltpu.VMEM((16 * NODE_W,), jnp.float32),
            pltpu.VMEM((16 * NODE_W,), jnp.float32),
            pltpu.VMEM((16 * EDGE_W,), jnp.float32),
            pltpu.VMEM((16 * EDGE_W,), jnp.float32),
            pltpu.VMEM((16 * NODE_W,), jnp.float32),
            pltpu.VMEM((16 * NODE_W,), jnp.float32),
            pltpu.VMEM((16 * EDGE_W,), jnp.float32),
            pltpu.VMEM((16 * EDGE_W,), jnp.float32),
            pltpu.VMEM((NODE_WP,), jnp.float32),
            pltpu.VMEM((NODE_WP,), jnp.float32),
            pltpu.VMEM((EDGE_W,), jnp.float32),
            pltpu.VMEM((EDGE_W,), jnp.float32),
            pltpu.VMEM((16,), jnp.float32),
            pltpu.VMEM((16,), jnp.float32),
            pltpu.SemaphoreType.DMA((2, 5)),
        ],
    )
    un_t = jnp.swapaxes(u_node, 1, 2).reshape(-1)
    ue_t = jnp.swapaxes(u_edge, 1, 2).reshape(-1)
    tlp, noh, eoh = call(un_t, ue_t, n_negc, n_lp, e_negc, e_lp)
    return (tlp,
            noh.reshape(B, MAX_SIZE, NODE_DIM),
            eoh.reshape(B, N_EDGES, BOND_DIM))


# speedup vs baseline: 2.8294x; 1.1102x over previous
import jax
import jax.numpy as jnp
from jax import lax
from jax.experimental import pallas as pl
from jax.experimental.pallas import tpu as pltpu
from jax.experimental.pallas import tpu_sc as plsc

MAX_SIZE = 38
NODE_DIM = 9
BOND_DIM = 4
N_EDGES = 378
B = 16384
NODE_W = MAX_SIZE * NODE_DIM
EDGE_W = N_EDGES * BOND_DIM
NODE_WP = 352

NW = 32
ROWS_W = B // NW
GROUPS = ROWS_W // 16

_LN2 = 0.6931471805599453
_SQRTH = 0.7071067811865476
_C3 = 2.0 / 3.0
_C5 = 2.0 / 5.0
_C7 = 2.0 / 7.0


def _log(u):
    bits = lax.bitcast_convert_type(u, jnp.int32)
    e = (bits >> 23) - 126
    m = lax.bitcast_convert_type(
        (bits & 0x007FFFFF) | 0x3F000000, jnp.float32)
    cond = m < _SQRTH
    m = jnp.where(cond, m + m, m)
    ef = (e - cond.astype(jnp.int32)).astype(jnp.float32)
    r = (m - 1.0) / (m + 1.0)
    r2 = r * r
    w = ((_C7 * r2 + _C5) * r2 + _C3) * r2 + 2.0
    return ef * _LN2 + r * w


def _argmin_step(j, s, best, bj):
    lt = s < best
    return jnp.where(lt, s, best), jnp.where(lt, jnp.int32(j), bj)


def _make_body(n_pos, n_cat, width, unroll):
    def body(ut, negc_t, lp_t, out_lp, out_oh,
             ub0, ub1, oh0, oh1, tnc, tnl, acc0, acc1, sems):
        wid = lax.axis_index("s") * 2 + lax.axis_index("c")

        pltpu.sync_copy(negc_t, tnc)
        pltpu.sync_copy(lp_t, tnl)

        lane = lax.iota(jnp.int32, 16)
        base = lane * width

        slots = ((ub0, oh0, acc0), (ub1, oh1, acc1))

        def in_copies(g, slot):
            ub = slots[slot][0]
            r0 = wid * ROWS_W + g * 16
            return (pltpu.make_async_copy(
                ut.at[pl.ds(r0 * width, 16 * width)], ub, sems.at[slot, 0]),)

        def out_copies(g, slot):
            _, oh, acc = slots[slot]
            r0 = wid * ROWS_W + g * 16
            return (pltpu.make_async_copy(
                        oh, out_oh.at[pl.ds(r0 * width, 16 * width)],
                        sems.at[slot, 1]),
                    pltpu.make_async_copy(
                        acc, out_lp.at[pl.ds(r0, 16)], sems.at[slot, 2]))

        def compute(g, slot):
            ub, oh, accb = slots[slot]

            def pos(p, acc):
                off = p * n_cat
                best = jnp.full((16,), jnp.float32(jnp.inf))
                bj = jnp.zeros((16,), jnp.int32)
                for j in range(n_cat):
                    u = plsc.load_gather(ub, [base + (j * n_pos + p)])
                    u = jnp.maximum(u, 1e-10)
                    negc = plsc.load_gather(
                        tnc, [jnp.full((16,), off + j, jnp.int32)])
                    s = _log(u) * negc
                    best, bj = _argmin_step(j, s, best, bj)
                ll = plsc.load_gather(tnl, [off + bj])
                for j in range(n_cat):
                    plsc.store_scatter(oh, [base + (off + j)],
                                       (bj == j).astype(jnp.float32))
                return acc + ll

            acc = plsc.parallel_loop(0, n_pos, unroll=unroll,
                                     carry=jnp.zeros((16,), jnp.float32))(pos)
            accb[...] = acc

        for c in in_copies(0, 0):
            c.start()

        def pair(p, _):
            for k in range(2):
                g = p * 2 + k

                @pl.when(g + 1 < GROUPS)
                def _():
                    for c in in_copies(g + 1, 1 - k):
                        c.start()

                for c in in_copies(g, k):
                    c.wait()

                @pl.when(g >= 2)
                def _():
                    for c in out_copies(g - 2, k):
                        c.wait()

                compute(g, k)
                for c in out_copies(g, k):
                    c.start()
            return 0

        lax.fori_loop(0, GROUPS // 2, pair, 0)
        for c in out_copies(GROUPS - 2, 0):
            c.wait()
        for c in out_copies(GROUPS - 1, 1):
            c.wait()

    return body


def _make_call(n_pos, n_cat, width, width_p, unroll):
    mesh = plsc.VectorSubcoreMesh(core_axis_name="c", subcore_axis_name="s")
    return pl.kernel(
        _make_body(n_pos, n_cat, width, unroll),
        out_type=[
            jax.ShapeDtypeStruct((B,), jnp.float32),
            jax.ShapeDtypeStruct((B * width,), jnp.float32),
        ],
        mesh=mesh,
        compiler_params=pltpu.CompilerParams(needs_layout_passes=False),
        scratch_types=[
            pltpu.VMEM((16 * width,), jnp.float32),
            pltpu.VMEM((16 * width,), jnp.float32),
            pltpu.VMEM((16 * width,), jnp.float32),
            pltpu.VMEM((16 * width,), jnp.float32),
            pltpu.VMEM((width_p,), jnp.float32),
            pltpu.VMEM((width_p,), jnp.float32),
            pltpu.VMEM((16,), jnp.float32),
            pltpu.VMEM((16,), jnp.float32),
            pltpu.SemaphoreType.DMA((2, 3)),
        ],
    )


@jax.jit
def kernel(u_node, u_edge, node_base_log_probs, edge_base_log_probs):
    nl = node_base_log_probs * 0.3
    el = edge_base_log_probs / 0.3
    n_negc = jnp.pad(-jnp.exp(-nl).reshape(-1), (0, NODE_WP - NODE_W),
                     constant_values=-1.0)
    n_lp = jnp.pad(jax.nn.log_softmax(nl, axis=-1).reshape(-1),
                   (0, NODE_WP - NODE_W))
    e_negc = -jnp.exp(-el).reshape(-1)
    e_lp = jax.nn.log_softmax(el, axis=-1).reshape(-1)

    un_t = jnp.swapaxes(u_node, 1, 2).reshape(-1)
    ue_t = jnp.swapaxes(u_edge, 1, 2).reshape(-1)

    node_call = _make_call(MAX_SIZE, NODE_DIM, NODE_W, NODE_WP, 2)
    edge_call = _make_call(N_EDGES, BOND_DIM, EDGE_W, EDGE_W, 6)

    nsum, noh = node_call(un_t, n_negc, n_lp)
    esum, eoh = edge_call(ue_t, e_negc, e_lp)

    return (nsum + esum,
            noh.reshape(B, MAX_SIZE, NODE_DIM),
            eoh.reshape(B, N_EDGES, BOND_DIM))


# speedup vs baseline: 3.5535x; 1.2559x over previous
import jax
import jax.numpy as jnp
from jax import lax
from jax.experimental import pallas as pl
from jax.experimental.pallas import tpu as pltpu
from jax.experimental.pallas import tpu_sc as plsc

MAX_SIZE = 38
NODE_DIM = 9
BOND_DIM = 4
N_EDGES = 378
B = 16384
NODE_W = MAX_SIZE * NODE_DIM
EDGE_W = N_EDGES * BOND_DIM
NODE_WP = 352

NW = 32
ECH = 4
BCH = B // ECH

_LN2 = 0.6931471805599453
_SQRTH = 0.7071067811865476
_C3 = 2.0 / 3.0
_C5 = 2.0 / 5.0
_C7 = 2.0 / 7.0


def _log(u):
    bits = lax.bitcast_convert_type(u, jnp.int32)
    e = (bits >> 23) - 126
    m = lax.bitcast_convert_type(
        (bits & 0x007FFFFF) | 0x3F000000, jnp.float32)
    cond = m < _SQRTH
    m = jnp.where(cond, m + m, m)
    ef = (e - cond.astype(jnp.int32)).astype(jnp.float32)
    r = (m - 1.0) / (m + 1.0)
    r2 = r * r
    w = ((_C7 * r2 + _C5) * r2 + _C3) * r2 + 2.0
    return ef * _LN2 + r * w


def _argmin_step(j, s, best, bj):
    lt = s < best
    return jnp.where(lt, s, best), jnp.where(lt, jnp.int32(j), bj)


def _make_body(n_pos, n_cat, width, unroll, rows):
    rows_w = rows // NW
    groups = rows_w // 16

    def body(ut, negc_t, lp_t, out_lp, out_oh,
             ub0, ub1, oh0, oh1, tnc, tnl, acc0, acc1, sems):
        wid = lax.axis_index("s") * 2 + lax.axis_index("c")

        pltpu.sync_copy(negc_t, tnc)
        pltpu.sync_copy(lp_t, tnl)

        lane = lax.iota(jnp.int32, 16)
        base = lane * width

        slots = ((ub0, oh0, acc0), (ub1, oh1, acc1))

        def in_copies(g, slot):
            ub = slots[slot][0]
            r0 = wid * rows_w + g * 16
            return (pltpu.make_async_copy(
                ut.at[pl.ds(r0 * width, 16 * width)], ub, sems.at[slot, 0]),)

        def out_copies(g, slot):
            _, oh, acc = slots[slot]
            r0 = wid * rows_w + g * 16
            return (pltpu.make_async_copy(
                        oh, out_oh.at[pl.ds(r0 * width, 16 * width)],
                        sems.at[slot, 1]),
                    pltpu.make_async_copy(
                        acc, out_lp.at[pl.ds(r0, 16)], sems.at[slot, 2]))

        def compute(g, slot):
            ub, oh, accb = slots[slot]

            def pos(p, acc):
                off = p * n_cat
                best = jnp.full((16,), jnp.float32(jnp.inf))
                bj = jnp.zeros((16,), jnp.int32)
                for j in range(n_cat):
                    u = plsc.load_gather(ub, [base + (j * n_pos + p)])
                    u = jnp.maximum(u, 1e-10)
                    negc = plsc.load_gather(
                        tnc, [jnp.full((16,), off + j, jnp.int32)])
                    s = _log(u) * negc
                    best, bj = _argmin_step(j, s, best, bj)
                ll = plsc.load_gather(tnl, [off + bj])
                for j in range(n_cat):
                    plsc.store_scatter(oh, [base + (off + j)],
                                       (bj == j).astype(jnp.float32))
                return acc + ll

            acc = plsc.parallel_loop(0, n_pos, unroll=unroll,
                                     carry=jnp.zeros((16,), jnp.float32))(pos)
            accb[...] = acc

        for c in in_copies(0, 0):
            c.start()

        def pair(p, _):
            for k in range(2):
                g = p * 2 + k

                @pl.when(g + 1 < groups)
                def _():
                    for c in in_copies(g + 1, 1 - k):
                        c.start()

                for c in in_copies(g, k):
                    c.wait()

                @pl.when(g >= 2)
                def _():
                    for c in out_copies(g - 2, k):
                        c.wait()

                compute(g, k)
                for c in out_copies(g, k):
                    c.start()
            return 0

        lax.fori_loop(0, groups // 2, pair, 0)
        for c in out_copies(groups - 2, 0):
            c.wait()
        for c in out_copies(groups - 1, 1):
            c.wait()

    return body


def _make_call(n_pos, n_cat, width, width_p, unroll, rows):
    mesh = plsc.VectorSubcoreMesh(core_axis_name="c", subcore_axis_name="s")
    return pl.kernel(
        _make_body(n_pos, n_cat, width, unroll, rows),
        out_type=[
            jax.ShapeDtypeStruct((rows,), jnp.float32),
            jax.ShapeDtypeStruct((rows * width,), jnp.float32),
        ],
        mesh=mesh,
        compiler_params=pltpu.CompilerParams(needs_layout_passes=False),
        scratch_types=[
            pltpu.VMEM((16 * width,), jnp.float32),
            pltpu.VMEM((16 * width,), jnp.float32),
            pltpu.VMEM((16 * width,), jnp.float32),
            pltpu.VMEM((16 * width,), jnp.float32),
            pltpu.VMEM((width_p,), jnp.float32),
            pltpu.VMEM((width_p,), jnp.float32),
            pltpu.VMEM((16,), jnp.float32),
            pltpu.VMEM((16,), jnp.float32),
            pltpu.SemaphoreType.DMA((2, 3)),
        ],
    )


@jax.jit
def kernel(u_node, u_edge, node_base_log_probs, edge_base_log_probs):
    nl = node_base_log_probs * 0.3
    el = edge_base_log_probs / 0.3
    n_negc = jnp.pad(-jnp.exp(-nl).reshape(-1), (0, NODE_WP - NODE_W),
                     constant_values=-1.0)
    n_lp = jnp.pad(jax.nn.log_softmax(nl, axis=-1).reshape(-1),
                   (0, NODE_WP - NODE_W))
    e_negc = -jnp.exp(-el).reshape(-1)
    e_lp = jax.nn.log_softmax(el, axis=-1).reshape(-1)

    un_t = jnp.swapaxes(u_node, 1, 2).reshape(-1)

    node_call = _make_call(MAX_SIZE, NODE_DIM, NODE_W, NODE_WP, 2, B)
    edge_call = _make_call(N_EDGES, BOND_DIM, EDGE_W, EDGE_W, 6, BCH)

    nsum, noh = node_call(un_t, n_negc, n_lp)
    esums, eohs = [], []
    for ch in range(ECH):
        ue_t = jnp.swapaxes(u_edge[ch * BCH:(ch + 1) * BCH], 1, 2).reshape(-1)
        es, eo = edge_call(ue_t, e_negc, e_lp)
        esums.append(es)
        eohs.append(eo.reshape(BCH, N_EDGES, BOND_DIM))

    return (nsum + jnp.concatenate(esums),
            noh.reshape(B, MAX_SIZE, NODE_DIM),
            jnp.concatenate(eohs, axis=0))
